# baseline (speedup 1.0000x reference)
import functools

import jax
import jax.numpy as jnp
from jax import lax
from jax.experimental import pallas as pl
from jax.experimental.pallas import tpu as pltpu

_B = 64
_U = 4


def _noiser_kernel(
    seg_time_ref,
    seg_start_ref,
    seg_cnt_ref,
    xs_ref,
    gs_ref,
    q_hbm_ref,
    probs_ref,
    noised_ref,
    q_vmem,
    q_sem,
):
    b = pl.program_id(0)
    B, C = xs_ref.shape

    @pl.when(b == 0)
    def _load_q():
        cp = pltpu.make_async_copy(q_hbm_ref, q_vmem, q_sem)
        cp.start()
        cp.wait()

    xb = xs_ref[...]
    rowid = lax.broadcasted_iota(jnp.int32, (B, 1), 0)

    def one_seg(s):
        t = seg_time_ref[b, s]
        st = seg_start_ref[b, s]
        en = seg_start_ref[b, s + 1]
        q = q_vmem[t]
        m = ((rowid >= st) & (rowid < en)).astype(jnp.float32)
        return jnp.dot(
            xb * m, q,
            preferred_element_type=jnp.float32,
            precision=lax.Precision.HIGHEST,
        )

    def body(i, accs):
        return tuple(
            acc + one_seg(i * _U + u) for u, acc in enumerate(accs)
        )

    nit = (seg_cnt_ref[b] + (_U - 1)) // _U
    zeros = jnp.zeros((B, C), jnp.float32)
    accs = lax.fori_loop(0, nit, body, (zeros,) * _U)
    probs = functools.reduce(lambda a, c: a + c, accs)
    probs_ref[...] = probs

    pn = probs / jnp.sum(probs, axis=-1, keepdims=True)
    score = gs_ref[...] + jnp.log(jnp.clip(pn, 1e-30, None))
    colid = lax.broadcasted_iota(jnp.int32, (B, C), 1)
    mx = jnp.max(score, axis=-1, keepdims=True)
    samp = jnp.min(jnp.where(score == mx, colid, C), axis=-1, keepdims=True)
    noised_ref[...] = (colid == samp).astype(jnp.float32)


@jax.jit
def kernel(x0_batch, time_batch, accumulated_q_matrices):
    N, C = x0_batch.shape
    T1 = accumulated_q_matrices.shape[0]
    B = _B
    NB = N // B

    t = time_batch.astype(jnp.int32)
    g = jax.random.gumbel(jax.random.key(1), (N, C), jnp.float32)

    perm = jnp.argsort(t)
    ts = t[perm]
    xs = x0_batch[perm]
    gs = g[perm]

    ts2 = ts.reshape(NB, B)
    prev = jnp.concatenate([ts2[:, :1] - 1, ts2[:, :-1]], axis=1)
    is_start = ts2 != prev
    seg_cnt = jnp.sum(is_start, axis=1).astype(jnp.int32)
    col = jnp.arange(B, dtype=jnp.int32)[None, :]
    starts = jnp.sort(jnp.where(is_start, col, B), axis=1).astype(jnp.int32)
    seg_time = jnp.take_along_axis(ts2, jnp.minimum(starts, B - 1), axis=1)
    starts_full = jnp.concatenate(
        [starts, jnp.full((NB, 1), B, jnp.int32)], axis=1
    )

    grid_spec = pltpu.PrefetchScalarGridSpec(
        num_scalar_prefetch=3,
        grid=(NB,),
        in_specs=[
            pl.BlockSpec((B, C), lambda b, *_: (b, 0)),
            pl.BlockSpec((B, C), lambda b, *_: (b, 0)),
            pl.BlockSpec(memory_space=pl.ANY),
        ],
        out_specs=[
            pl.BlockSpec((B, C), lambda b, *_: (b, 0)),
            pl.BlockSpec((B, C), lambda b, *_: (b, 0)),
        ],
        scratch_shapes=[
            pltpu.VMEM((T1, C, C), jnp.float32),
            pltpu.SemaphoreType.DMA,
        ],
    )

    probs_s, noised_s = pl.pallas_call(
        _noiser_kernel,
        grid_spec=grid_spec,
        out_shape=[
            jax.ShapeDtypeStruct((N, C), jnp.float32),
            jax.ShapeDtypeStruct((N, C), jnp.float32),
        ],
    )(seg_time, starts_full, seg_cnt, xs, gs, accumulated_q_matrices)

    probs = jnp.zeros((N, C), jnp.float32).at[perm].set(probs_s)
    noised = jnp.zeros((N, C), jnp.float32).at[perm].set(noised_s)
    return probs, noised


# speedup vs baseline: 1.1307x; 1.1307x over previous
import functools

import jax
import jax.numpy as jnp
from jax import lax
from jax.experimental import pallas as pl
from jax.experimental.pallas import tpu as pltpu

_B = 128
_U = 4


def _noiser_kernel(
    seg_time_ref,
    seg_start_ref,
    seg_cnt_ref,
    xs_ref,
    gs_ref,
    q_vmem,
    probs_ref,
    noised_ref,
):
    b = pl.program_id(0)
    B, C = xs_ref.shape

    xb = xs_ref[...]
    rowid = lax.broadcasted_iota(jnp.int32, (B, 1), 0)

    def one_seg(s):
        t = seg_time_ref[b, s]
        st = seg_start_ref[b, s]
        en = seg_start_ref[b, s + 1]
        q = q_vmem[t]
        m = ((rowid >= st) & (rowid < en)).astype(jnp.float32)
        return jnp.dot(
            xb * m, q,
            preferred_element_type=jnp.float32,
            precision=lax.Precision.HIGHEST,
        )

    def body(i, accs):
        return tuple(
            acc + one_seg(i * _U + u) for u, acc in enumerate(accs)
        )

    nit = (seg_cnt_ref[b] + (_U - 1)) // _U
    zeros = jnp.zeros((B, C), jnp.float32)
    accs = lax.fori_loop(0, nit, body, (zeros,) * _U)
    probs = functools.reduce(lambda a, c: a + c, accs)
    probs_ref[...] = probs

    pn = probs / jnp.sum(probs, axis=-1, keepdims=True)
    score = gs_ref[...] + jnp.log(jnp.clip(pn, 1e-30, None))
    colid = lax.broadcasted_iota(jnp.int32, (B, C), 1)
    mx = jnp.max(score, axis=-1, keepdims=True)
    samp = jnp.min(jnp.where(score == mx, colid, C), axis=-1, keepdims=True)
    noised_ref[...] = (colid == samp).astype(jnp.float32)


@jax.jit
def kernel(x0_batch, time_batch, accumulated_q_matrices):
    N, C = x0_batch.shape
    T1 = accumulated_q_matrices.shape[0]
    B = _B
    NB = N // B

    t = time_batch.astype(jnp.int32)
    g = jax.random.gumbel(jax.random.key(1), (N, C), jnp.float32)

    perm = jnp.argsort(t)
    ts = t[perm]
    xs = x0_batch[perm]
    gs = g[perm]

    ts2 = ts.reshape(NB, B)
    prev = jnp.concatenate([ts2[:, :1] - 1, ts2[:, :-1]], axis=1)
    is_start = ts2 != prev
    seg_cnt = jnp.sum(is_start, axis=1).astype(jnp.int32)
    col = jnp.arange(B, dtype=jnp.int32)[None, :]
    starts = jnp.sort(jnp.where(is_start, col, B), axis=1).astype(jnp.int32)
    seg_time = jnp.take_along_axis(ts2, jnp.minimum(starts, B - 1), axis=1)
    starts_full = jnp.concatenate(
        [starts, jnp.full((NB, 1), B, jnp.int32)], axis=1
    )

    grid_spec = pltpu.PrefetchScalarGridSpec(
        num_scalar_prefetch=3,
        grid=(NB,),
        in_specs=[
            pl.BlockSpec((B, C), lambda b, *_: (b, 0)),
            pl.BlockSpec((B, C), lambda b, *_: (b, 0)),
            pl.BlockSpec(
                (accumulated_q_matrices.shape[0], 100, 100),
                lambda b, *_: (0, 0, 0),
            ),
        ],
        out_specs=[
            pl.BlockSpec((B, C), lambda b, *_: (b, 0)),
            pl.BlockSpec((B, C), lambda b, *_: (b, 0)),
        ],
    )

    probs_s, noised_s = pl.pallas_call(
        _noiser_kernel,
        grid_spec=grid_spec,
        out_shape=[
            jax.ShapeDtypeStruct((N, C), jnp.float32),
            jax.ShapeDtypeStruct((N, C), jnp.float32),
        ],
    )(seg_time, starts_full, seg_cnt, xs, gs, accumulated_q_matrices)

    probs = jnp.zeros((N, C), jnp.float32).at[perm].set(probs_s)
    noised = jnp.zeros((N, C), jnp.float32).at[perm].set(noised_s)
    return probs, noised


# speedup vs baseline: 1.1741x; 1.0383x over previous
import functools

import jax
import jax.numpy as jnp
from jax import lax
from jax.experimental import pallas as pl
from jax.experimental.pallas import tpu as pltpu

_B = 128
_U = 5


def _noiser_kernel(
    seg_time_ref,
    seg_start_ref,
    seg_cnt_ref,
    xs_ref,
    gs_ref,
    q_vmem,
    probs_ref,
    noised_ref,
):
    b = pl.program_id(0)
    B, C = xs_ref.shape

    xb = xs_ref[...]
    rowid = lax.broadcasted_iota(jnp.int32, (B, 1), 0)

    def one_seg(s):
        t = seg_time_ref[b, s]
        st = seg_start_ref[b, s]
        en = seg_start_ref[b, s + 1]
        q = q_vmem[t]
        m = ((rowid >= st) & (rowid < en)).astype(jnp.float32)
        return jnp.dot(
            xb * m, q,
            preferred_element_type=jnp.float32,
            precision=lax.Precision.HIGHEST,
        )

    def body(i, accs):
        return tuple(
            acc + one_seg(i * _U + u) for u, acc in enumerate(accs)
        )

    nit = (seg_cnt_ref[b] + (_U - 1)) // _U
    zeros = jnp.zeros((B, C), jnp.float32)
    accs = lax.fori_loop(0, nit, body, (zeros,) * _U)
    probs = functools.reduce(lambda a, c: a + c, accs)
    probs_ref[...] = probs

    pn = probs / jnp.sum(probs, axis=-1, keepdims=True)
    score = gs_ref[...] + jnp.log(jnp.clip(pn, 1e-30, None))
    colid = lax.broadcasted_iota(jnp.int32, (B, C), 1)
    mx = jnp.max(score, axis=-1, keepdims=True)
    samp = jnp.min(jnp.where(score == mx, colid, C), axis=-1, keepdims=True)
    noised_ref[...] = (colid == samp).astype(jnp.float32)


@jax.jit
def kernel(x0_batch, time_batch, accumulated_q_matrices):
    N, C = x0_batch.shape
    T1 = accumulated_q_matrices.shape[0]
    B = _B
    NB = N // B

    t = time_batch.astype(jnp.int32)
    g = jax.random.gumbel(jax.random.key(1), (N, C), jnp.float32)

    perm = jnp.argsort(t)
    ts = t[perm]
    xs = x0_batch[perm]
    gs = g[perm]

    ts2 = ts.reshape(NB, B)
    prev = jnp.concatenate([ts2[:, :1] - 1, ts2[:, :-1]], axis=1)
    is_start = ts2 != prev
    seg_cnt = jnp.sum(is_start, axis=1).astype(jnp.int32)
    col = jnp.arange(B, dtype=jnp.int32)[None, :]
    starts = jnp.sort(jnp.where(is_start, col, B), axis=1).astype(jnp.int32)
    seg_time = jnp.take_along_axis(ts2, jnp.minimum(starts, B - 1), axis=1)
    PW = ((B + _U - 1) // _U) * _U
    seg_time = jnp.concatenate(
        [seg_time, jnp.zeros((NB, PW - B), jnp.int32)], axis=1
    )
    starts_full = jnp.concatenate(
        [starts, jnp.full((NB, PW + 1 - B), B, jnp.int32)], axis=1
    )

    grid_spec = pltpu.PrefetchScalarGridSpec(
        num_scalar_prefetch=3,
        grid=(NB,),
        in_specs=[
            pl.BlockSpec((B, C), lambda b, *_: (b, 0)),
            pl.BlockSpec((B, C), lambda b, *_: (b, 0)),
            pl.BlockSpec(
                (accumulated_q_matrices.shape[0], 100, 100),
                lambda b, *_: (0, 0, 0),
            ),
        ],
        out_specs=[
            pl.BlockSpec((B, C), lambda b, *_: (b, 0)),
            pl.BlockSpec((B, C), lambda b, *_: (b, 0)),
        ],
    )

    probs_s, noised_s = pl.pallas_call(
        _noiser_kernel,
        grid_spec=grid_spec,
        out_shape=[
            jax.ShapeDtypeStruct((N, C), jnp.float32),
            jax.ShapeDtypeStruct((N, C), jnp.float32),
        ],
    )(seg_time, starts_full, seg_cnt, xs, gs, accumulated_q_matrices)

    probs = jnp.zeros((N, C), jnp.float32).at[perm].set(probs_s)
    noised = jnp.zeros((N, C), jnp.float32).at[perm].set(noised_s)
    return probs, noised
